# baseline (speedup 1.0000x reference)
import jax
import jax.numpy as jnp
from jax.experimental import pallas as pl

_B, _T = 16, 16000
_N_IN, _N_OUT = 256, 512
_TT = 3200
_NT = _T // _TT
_BT = 4


def _onehot_kernel(x_ref, tab_ref, out_ref):
    tab = tab_ref[...]
    ss = (tab / (1.0 + jnp.abs(tab))).astype(jnp.bfloat16)
    iota = jax.lax.broadcasted_iota(jnp.int32, (_N_IN, _TT), 0)
    for j in range(_BT):
        idx = x_ref[j, 0, 0, :]
        oh = (iota == idx[None, :]).astype(jnp.bfloat16)
        out_ref[j, :, :] = jax.lax.dot_general(
            ss, oh,
            (((0,), (0,)), ((), ())),
            preferred_element_type=jnp.float32,
        )


def _lookup(x, table):
    b = x.shape[0]
    x4 = x.astype(jnp.int32).reshape(b, _NT, 1, _TT)
    return pl.pallas_call(
        _onehot_kernel,
        grid=(b // _BT, _NT),
        in_specs=[
            pl.BlockSpec((_BT, 1, 1, _TT), lambda i, t: (i, t, 0, 0)),
            pl.BlockSpec((_N_IN, _N_OUT), lambda i, t: (0, 0)),
        ],
        out_specs=pl.BlockSpec((_BT, _N_OUT, _TT), lambda i, t: (i, 0, t)),
        out_shape=jax.ShapeDtypeStruct((b, _N_OUT, _T), jnp.float32),
    )(x4, table)


def kernel(x, table):
    return _lookup(x, table)


# speedup vs baseline: 1.0133x; 1.0133x over previous
import jax
import jax.numpy as jnp
from jax.experimental import pallas as pl

_B, _T = 16, 16000
_N_IN, _N_OUT = 256, 512
_CC = 256
_C2 = _N_OUT // _CC
_TC = 3200
_NC = _T // _TC


def _onehot_kernel(x_ref, tab_ref, out_ref):
    tab = tab_ref[...]
    ss = (tab / (1.0 + jnp.abs(tab))).astype(jnp.bfloat16)
    iota = jax.lax.broadcasted_iota(jnp.int32, (_N_IN, _TC), 0)
    for n in range(_NC):
        idx = x_ref[0, 0, n * _TC:(n + 1) * _TC]
        oh = (iota == idx[None, :]).astype(jnp.bfloat16)
        out_ref[0, :, n * _TC:(n + 1) * _TC] = jax.lax.dot_general(
            ss, oh,
            (((0,), (0,)), ((), ())),
            preferred_element_type=jnp.float32,
        )


def _lookup(x, table):
    b = x.shape[0]
    x3 = x.astype(jnp.int32).reshape(b, 1, _T)
    return pl.pallas_call(
        _onehot_kernel,
        grid=(b, _C2),
        in_specs=[
            pl.BlockSpec((1, 1, _T), lambda i, c: (i, 0, 0)),
            pl.BlockSpec((_N_IN, _CC), lambda i, c: (0, c)),
        ],
        out_specs=pl.BlockSpec((1, _CC, _T), lambda i, c: (i, c, 0)),
        out_shape=jax.ShapeDtypeStruct((b, _N_OUT, _T), jnp.float32),
    )(x3, table)


def kernel(x, table):
    return _lookup(x, table)
